# baseline (speedup 1.0000x reference)
import functools

import jax
import jax.numpy as jnp
from jax import lax
from jax.experimental import pallas as pl
from jax.experimental.pallas import tpu as pltpu
from jax.experimental.pallas import tpu_sc as plsc

MAX_REL = 128
NC = 2
NS = 16
LANES = 16


def _tc_body(table_ref, out_ref, buf, sems, *, S, D, BB, B, NSEM):
    T = table_ref.shape[0]
    i = jax.lax.broadcasted_iota(jnp.int32, (S, T), 0)
    t = jax.lax.broadcasted_iota(jnp.int32, (S, T), 1)
    t_lo = jnp.maximum(i - (S - 1 - MAX_REL), 0)
    t_hi = jnp.minimum(i + MAX_REL, 2 * MAX_REL)
    band = ((t >= t_lo) & (t <= t_hi)).astype(jnp.float32)
    lo_extra = jnp.where(t == 0, jnp.maximum((S - 1 - MAX_REL) - i, 0), 0)
    hi_extra = jnp.where(t == 2 * MAX_REL, jnp.maximum(i - MAX_REL, 0), 0)
    counts = band + lo_extra.astype(jnp.float32) + hi_extra.astype(jnp.float32)
    summed = jnp.dot(counts, table_ref[...], preferred_element_type=jnp.float32)
    buf[...] = jnp.broadcast_to(summed[None, :, :], (BB, S, D))

    nblk = B // BB
    for k in range(nblk):
        if k >= NSEM:
            pltpu.make_async_copy(
                buf,
                out_ref.at[pl.ds((k - NSEM) * BB, BB)],
                sems.at[(k - NSEM) % NSEM],
            ).wait()
        pltpu.make_async_copy(
            buf, out_ref.at[pl.ds(k * BB, BB)], sems.at[k % NSEM]
        ).start()
    for k in range(max(nblk - NSEM, 0), nblk):
        pltpu.make_async_copy(
            buf, out_ref.at[pl.ds(k * BB, BB)], sems.at[k % NSEM]
        ).wait()


def _tc_broadcast(table, B, S, D):
    BB = 16
    NSEM = 32
    return pl.pallas_call(
        functools.partial(_tc_body, S=S, D=D, BB=BB, B=B, NSEM=NSEM),
        in_specs=[pl.BlockSpec(memory_space=pltpu.MemorySpace.VMEM)],
        out_specs=pl.BlockSpec(memory_space=pl.ANY),
        out_shape=jax.ShapeDtypeStruct((B, S, D), jnp.float32),
        scratch_shapes=[
            pltpu.VMEM((BB, S, D), jnp.float32),
            pltpu.SemaphoreType.DMA((NSEM,)),
        ],
    )(table)


def _sc_body(table_hbm, out_hbm, table_v, rows_v, *, S, D, T, NROWS):
    sid = lax.axis_index("s")
    NCH = D // LANES

    pltpu.sync_copy(table_hbm, table_v)

    i0 = jnp.minimum(sid * NROWS, S - NROWS)

    a = jnp.maximum(i0 - (S - 1 - MAX_REL), 0)
    bnd = jnp.minimum(i0 + MAX_REL, 2 * MAX_REL)

    def win_body(t, acc):
        return tuple(
            acc[c] + table_v[t, pl.ds(c * LANES, LANES)] for c in range(NCH)
        )

    acc = tuple(jnp.zeros((LANES,), jnp.float32) for _ in range(NCH))
    acc = lax.fori_loop(a, bnd + 1, win_body, acc)
    lo_f = jnp.maximum((S - 1 - MAX_REL) - i0, 0).astype(jnp.float32)
    hi_f = jnp.maximum(i0 - MAX_REL, 0).astype(jnp.float32)
    acc = tuple(
        acc[c]
        + lo_f * table_v[0, pl.ds(c * LANES, LANES)]
        + hi_f * table_v[2 * MAX_REL, pl.ds(c * LANES, LANES)]
        for c in range(NCH)
    )
    for c in range(NCH):
        rows_v[0, pl.ds(c * LANES, LANES)] = acc[c]

    for r in range(1, NROWS):
        i = i0 + r
        add_t = jnp.minimum(i + MAX_REL, 2 * MAX_REL)
        sub_t = jnp.maximum(i - (S - 1 - MAX_REL) - 1, 0)
        acc = tuple(
            acc[c]
            + table_v[add_t, pl.ds(c * LANES, LANES)]
            - table_v[sub_t, pl.ds(c * LANES, LANES)]
            for c in range(NCH)
        )
        for c in range(NCH):
            rows_v[r, pl.ds(c * LANES, LANES)] = acc[c]

    pltpu.sync_copy(rows_v, out_hbm.at[pl.ds(i0, NROWS)])


def _sc_summed(table, S, D):
    T = table.shape[0]
    NROWS = 16
    mesh = plsc.VectorSubcoreMesh(core_axis_name="c", subcore_axis_name="s")
    kfn = pl.kernel(
        functools.partial(_sc_body, S=S, D=D, T=T, NROWS=NROWS),
        mesh=mesh,
        out_type=jax.ShapeDtypeStruct((S, D), jnp.float32),
        scratch_types=[
            pltpu.VMEM((T, D), jnp.float32),
            pltpu.VMEM((NROWS, D), jnp.float32),
        ],
    )
    return kfn(table)


def kernel(inputs, table):
    B, S = inputs.shape
    T, D = table.shape
    return _tc_broadcast(table, B, S, D)


# speedup vs baseline: 1.0069x; 1.0069x over previous
import functools

import jax
import jax.numpy as jnp
from jax import lax
from jax.experimental import pallas as pl
from jax.experimental.pallas import tpu as pltpu
from jax.experimental.pallas import tpu_sc as plsc

MAX_REL = 128
NC = 2
NS = 16
LANES = 16


def _tc_body(table_ref, out_ref, buf, sems, *, S, D, BB, B, NSEM):
    T = table_ref.shape[0]
    i = jax.lax.broadcasted_iota(jnp.int32, (S, T), 0)
    t = jax.lax.broadcasted_iota(jnp.int32, (S, T), 1)
    t_lo = jnp.maximum(i - (S - 1 - MAX_REL), 0)
    t_hi = jnp.minimum(i + MAX_REL, 2 * MAX_REL)
    band = ((t >= t_lo) & (t <= t_hi)).astype(jnp.float32)
    lo_extra = jnp.where(t == 0, jnp.maximum((S - 1 - MAX_REL) - i, 0), 0)
    hi_extra = jnp.where(t == 2 * MAX_REL, jnp.maximum(i - MAX_REL, 0), 0)
    counts = band + lo_extra.astype(jnp.float32) + hi_extra.astype(jnp.float32)
    summed = jnp.dot(counts, table_ref[...], preferred_element_type=jnp.float32)
    buf[...] = jnp.broadcast_to(summed[None, :, :], (BB, S, D))

    nblk = B // BB
    for k in range(nblk):
        if k >= NSEM:
            pltpu.make_async_copy(
                buf,
                out_ref.at[pl.ds((k - NSEM) * BB, BB)],
                sems.at[(k - NSEM) % NSEM],
            ).wait()
        pltpu.make_async_copy(
            buf, out_ref.at[pl.ds(k * BB, BB)], sems.at[k % NSEM]
        ).start()
    for k in range(max(nblk - NSEM, 0), nblk):
        pltpu.make_async_copy(
            buf, out_ref.at[pl.ds(k * BB, BB)], sems.at[k % NSEM]
        ).wait()


def _tc_broadcast(table, B, S, D):
    BB = 4
    NSEM = 64
    return pl.pallas_call(
        functools.partial(_tc_body, S=S, D=D, BB=BB, B=B, NSEM=NSEM),
        in_specs=[pl.BlockSpec(memory_space=pltpu.MemorySpace.VMEM)],
        out_specs=pl.BlockSpec(memory_space=pl.ANY),
        out_shape=jax.ShapeDtypeStruct((B, S, D), jnp.float32),
        scratch_shapes=[
            pltpu.VMEM((BB, S, D), jnp.float32),
            pltpu.SemaphoreType.DMA((NSEM,)),
        ],
    )(table)


def _sc_body(table_hbm, out_hbm, table_v, rows_v, *, S, D, T, NROWS):
    sid = lax.axis_index("s")
    NCH = D // LANES

    pltpu.sync_copy(table_hbm, table_v)

    i0 = jnp.minimum(sid * NROWS, S - NROWS)

    a = jnp.maximum(i0 - (S - 1 - MAX_REL), 0)
    bnd = jnp.minimum(i0 + MAX_REL, 2 * MAX_REL)

    def win_body(t, acc):
        return tuple(
            acc[c] + table_v[t, pl.ds(c * LANES, LANES)] for c in range(NCH)
        )

    acc = tuple(jnp.zeros((LANES,), jnp.float32) for _ in range(NCH))
    acc = lax.fori_loop(a, bnd + 1, win_body, acc)
    lo_f = jnp.maximum((S - 1 - MAX_REL) - i0, 0).astype(jnp.float32)
    hi_f = jnp.maximum(i0 - MAX_REL, 0).astype(jnp.float32)
    acc = tuple(
        acc[c]
        + lo_f * table_v[0, pl.ds(c * LANES, LANES)]
        + hi_f * table_v[2 * MAX_REL, pl.ds(c * LANES, LANES)]
        for c in range(NCH)
    )
    for c in range(NCH):
        rows_v[0, pl.ds(c * LANES, LANES)] = acc[c]

    for r in range(1, NROWS):
        i = i0 + r
        add_t = jnp.minimum(i + MAX_REL, 2 * MAX_REL)
        sub_t = jnp.maximum(i - (S - 1 - MAX_REL) - 1, 0)
        acc = tuple(
            acc[c]
            + table_v[add_t, pl.ds(c * LANES, LANES)]
            - table_v[sub_t, pl.ds(c * LANES, LANES)]
            for c in range(NCH)
        )
        for c in range(NCH):
            rows_v[r, pl.ds(c * LANES, LANES)] = acc[c]

    pltpu.sync_copy(rows_v, out_hbm.at[pl.ds(i0, NROWS)])


def _sc_summed(table, S, D):
    T = table.shape[0]
    NROWS = 16
    mesh = plsc.VectorSubcoreMesh(core_axis_name="c", subcore_axis_name="s")
    kfn = pl.kernel(
        functools.partial(_sc_body, S=S, D=D, T=T, NROWS=NROWS),
        mesh=mesh,
        out_type=jax.ShapeDtypeStruct((S, D), jnp.float32),
        scratch_types=[
            pltpu.VMEM((T, D), jnp.float32),
            pltpu.VMEM((NROWS, D), jnp.float32),
        ],
    )
    return kfn(table)


def kernel(inputs, table):
    B, S = inputs.shape
    T, D = table.shape
    return _tc_broadcast(table, B, S, D)


# speedup vs baseline: 1.0128x; 1.0058x over previous
import functools

import jax
import jax.numpy as jnp
from jax.experimental import pallas as pl
from jax.experimental.pallas import tpu as pltpu

MAX_REL = 128


def _rel_emb_body(table_ref, out_ref, buf, sems, *, S, D, BB, B, NSEM):
    T = table_ref.shape[0]

    i = jax.lax.broadcasted_iota(jnp.int32, (S, T), 0)
    t = jax.lax.broadcasted_iota(jnp.int32, (S, T), 1)
    t_lo = jnp.maximum(i - (S - 1 - MAX_REL), 0)
    t_hi = jnp.minimum(i + MAX_REL, 2 * MAX_REL)
    band = ((t >= t_lo) & (t <= t_hi)).astype(jnp.float32)
    lo_extra = jnp.where(t == 0, jnp.maximum((S - 1 - MAX_REL) - i, 0), 0)
    hi_extra = jnp.where(t == 2 * MAX_REL, jnp.maximum(i - MAX_REL, 0), 0)
    counts = band + lo_extra.astype(jnp.float32) + hi_extra.astype(jnp.float32)
    summed = jnp.dot(counts, table_ref[...], preferred_element_type=jnp.float32)
    buf[...] = jnp.broadcast_to(summed[None, :, :], (BB, S, D))

    nblk = B // BB
    for k in range(nblk):
        if k >= NSEM:
            pltpu.make_async_copy(
                buf,
                out_ref.at[pl.ds((k - NSEM) * BB, BB)],
                sems.at[(k - NSEM) % NSEM],
            ).wait()
        pltpu.make_async_copy(
            buf, out_ref.at[pl.ds(k * BB, BB)], sems.at[k % NSEM]
        ).start()
    for k in range(max(nblk - NSEM, 0), nblk):
        pltpu.make_async_copy(
            buf, out_ref.at[pl.ds(k * BB, BB)], sems.at[k % NSEM]
        ).wait()


def kernel(inputs, table):
    B, S = inputs.shape
    T, D = table.shape
    BB = 8
    NSEM = 32
    return pl.pallas_call(
        functools.partial(_rel_emb_body, S=S, D=D, BB=BB, B=B, NSEM=NSEM),
        in_specs=[pl.BlockSpec(memory_space=pltpu.MemorySpace.VMEM)],
        out_specs=pl.BlockSpec(memory_space=pl.ANY),
        out_shape=jax.ShapeDtypeStruct((B, S, D), jnp.float32),
        scratch_shapes=[
            pltpu.VMEM((BB, S, D), jnp.float32),
            pltpu.SemaphoreType.DMA((NSEM,)),
        ],
    )(table)
